# Initial kernel scaffold; baseline (speedup 1.0000x reference)
import jax
import jax.numpy as jnp
from jax.experimental import pallas as pl


def kernel(x, edge_index, batch, W1, b1, W2, b2, Wp, bp, input_scale, quantum_scale, qweights, Wc1, bc1, Wc2, bc2):
    raise NotImplementedError("write your pallas kernel here")


# speedup vs baseline: 42.9983x; 42.9983x over previous
import functools

import numpy as np
import jax
import jax.numpy as jnp
from jax import lax
from jax.experimental import pallas as pl
from jax.experimental.pallas import tpu as pltpu
from jax.experimental.pallas import tpu_sc as plsc

N_NODES = 10000
D_FEAT = 128
N_GRAPHS = 128
N_QUBITS = 4
N_LAYERS = 2

NC, NS, LANES = 2, 16, 16
NW = NC * NS
NPAD = 10240
RPT = NPAD // NS
E = 320000
CH = 128
EPW = 10240
NCHUNK = EPW // CH
EP = NW * EPW
NBUF = 4


def _zero_block(zb, d):
    @pl.loop(0, CH)
    def _(r):
        for j in range(d // LANES):
            zb[r, pl.ds(j * LANES, LANES)] = jnp.zeros((LANES,), jnp.float32)


def _fill_ones(zb, d):
    @pl.loop(0, CH)
    def _(r):
        for j in range(d // LANES):
            zb[r, pl.ds(j * LANES, LANES)] = jnp.ones((LANES,), jnp.float32)


_MESH = plsc.VectorSubcoreMesh(
    core_axis_name="c", subcore_axis_name="s", num_cores=NC, num_subcores=NS)
_SC_PARAMS = pltpu.CompilerParams(use_tc_tiling_on_sc=False)


def _make_deg_kernel():
    d = 16

    @functools.partial(
        pl.kernel,
        out_type=jax.ShapeDtypeStruct((NC, NPAD, d), jnp.float32),
        mesh=_MESH,
        compiler_params=_SC_PARAMS,
        scratch_types=[
            pltpu.VMEM((NCHUNK, CH), jnp.int32),
            pltpu.VMEM((CH, d), jnp.float32),
            pltpu.VMEM((CH, d), jnp.float32),
            pltpu.VMEM_SHARED((NPAD, d), jnp.float32),
        ],
    )
    def deg_kernel(dst_hbm, out_hbm, dst_v, ones_v, zb_v, acc_sh):
        c = lax.axis_index("c")
        s = lax.axis_index("s")
        wid = c * NS + s
        _zero_block(zb_v, d)
        _fill_ones(ones_v, d)
        for k in range(RPT // CH):
            pltpu.sync_copy(zb_v, acc_sh.at[pl.ds(s * RPT + k * CH, CH)])
        pltpu.sync_copy(dst_hbm.at[wid], dst_v)
        plsc.subcore_barrier()

        @pl.loop(0, NCHUNK)
        def _(t):
            pltpu.sync_copy(ones_v, acc_sh.at[dst_v.at[t]], add=True)

        plsc.subcore_barrier()
        for k in range(RPT // CH):
            off = s * RPT + k * CH
            pltpu.sync_copy(acc_sh.at[pl.ds(off, CH)],
                            out_hbm.at[c, pl.ds(off, CH)])

    return deg_kernel


def _make_scatter_kernel(d):
    @functools.partial(
        pl.kernel,
        out_type=jax.ShapeDtypeStruct((NC, NPAD, d), jnp.float32),
        mesh=_MESH,
        compiler_params=_SC_PARAMS,
        scratch_types=[
            pltpu.VMEM((NCHUNK, CH), jnp.int32),
            pltpu.VMEM((NCHUNK, CH), jnp.int32),
            pltpu.VMEM((NBUF, CH, d), jnp.float32),
            pltpu.VMEM((CH, d), jnp.float32),
            pltpu.VMEM_SHARED((NPAD, d), jnp.float32),
        ] + [pltpu.SemaphoreType.DMA] * NBUF,
    )
    def scatter_kernel(g_hbm, src_hbm, dst_hbm, out_hbm,
                       src_v, dst_v, rows_v, zb_v, acc_sh, *sems):
        c = lax.axis_index("c")
        s = lax.axis_index("s")
        wid = c * NS + s
        _zero_block(zb_v, d)
        for k in range(RPT // CH):
            pltpu.sync_copy(zb_v, acc_sh.at[pl.ds(s * RPT + k * CH, CH)])
        pltpu.sync_copy(src_hbm.at[wid], src_v)
        pltpu.sync_copy(dst_hbm.at[wid], dst_v)
        plsc.subcore_barrier()

        @pl.loop(0, NCHUNK, step=NBUF)
        def _(t0):
            descs = []
            for b in range(NBUF):
                descs.append(pltpu.async_copy(
                    g_hbm.at[src_v.at[t0 + b]], rows_v.at[b], sems[b]))
            for b in range(NBUF):
                descs[b].wait()
                pltpu.sync_copy(rows_v.at[b], acc_sh.at[dst_v.at[t0 + b]],
                                add=True)

        plsc.subcore_barrier()
        for k in range(RPT // CH):
            off = s * RPT + k * CH
            pltpu.sync_copy(acc_sh.at[pl.ds(off, CH)],
                            out_hbm.at[c, pl.ds(off, CH)])

    return scatter_kernel


_deg_call = _make_deg_kernel()
_scat32 = _make_scatter_kernel(32)
_scat16 = _make_scatter_kernel(16)


def _dis_from(degp_ref):
    deg = degp_ref[0, :, 0:1] + degp_ref[1, :, 0:1] + 1.0
    return lax.rsqrt(deg)


def _tc1_body(x_ref, w1_ref, degp_ref, g1_ref):
    dis = _dis_from(degp_ref)
    h = jnp.dot(x_ref[...], w1_ref[...], preferred_element_type=jnp.float32)
    g1_ref[...] = h * dis


def _tc2_body(accp_ref, g1_ref, degp_ref, b1_ref, w2_ref, g2_ref):
    dis = _dis_from(degp_ref)
    z = (accp_ref[0] + accp_ref[1] + g1_ref[...]) * dis + b1_ref[...]
    h1 = jnp.maximum(z, 0.0)
    g2_ref[...] = jnp.dot(h1, w2_ref[...],
                          preferred_element_type=jnp.float32) * dis


def _tc3_body(accp_ref, g2_ref, degp_ref, b2_ref, batch_ref, pooled_ref):
    dis = _dis_from(degp_ref)
    z = (accp_ref[0] + accp_ref[1] + g2_ref[...]) * dis + b2_ref[...]
    h2 = jnp.maximum(z, 0.0)
    gi = lax.broadcasted_iota(jnp.int32, (N_GRAPHS, NPAD), 0)
    onehot = (batch_ref[...] == gi).astype(jnp.float32)
    sums = jnp.dot(onehot, h2, preferred_element_type=jnp.float32,
                   precision=lax.Precision.HIGHEST)
    counts = jnp.sum(onehot, axis=1, keepdims=True)
    pooled_ref[...] = sums / jnp.maximum(counts, 1.0)


_tc1 = pl.pallas_call(
    _tc1_body, out_shape=jax.ShapeDtypeStruct((NPAD, 32), jnp.float32))
_tc2 = pl.pallas_call(
    _tc2_body, out_shape=jax.ShapeDtypeStruct((NPAD, 16), jnp.float32))
_tc3 = pl.pallas_call(
    _tc3_body, out_shape=jax.ShapeDtypeStruct((N_GRAPHS, 16), jnp.float32))


def _q_apply_1q(state, u, wire):
    s = state.reshape((2,) * N_QUBITS)
    s = jnp.moveaxis(s, wire, 0)
    s = jnp.tensordot(u, s, axes=([1], [0]))
    s = jnp.moveaxis(s, 0, wire)
    return s.reshape(-1)


def _q_cnot(state, c, t):
    s = state.reshape((2,) * N_QUBITS)
    s = jnp.moveaxis(s, (c, t), (0, 1))
    s = jnp.stack([s[0], s[1][::-1]], axis=0)
    s = jnp.moveaxis(s, (0, 1), (c, t))
    return s.reshape(-1)


def _q_ry(t):
    c = jnp.cos(t / 2).astype(jnp.complex64)
    sn = jnp.sin(t / 2).astype(jnp.complex64)
    return jnp.stack([jnp.stack([c, -sn]), jnp.stack([sn, c])])


def _q_rz(t):
    p = jnp.exp(-0.5j * t).astype(jnp.complex64)
    m = jnp.exp(0.5j * t).astype(jnp.complex64)
    z = jnp.zeros((), jnp.complex64)
    return jnp.stack([jnp.stack([p, z]), jnp.stack([z, m])])


def _q_circuit(inputs, weights):
    s = jnp.zeros((2 ** N_QUBITS,), jnp.complex64).at[0].set(1.0)
    for i in range(N_QUBITS):
        s = _q_apply_1q(s, _q_ry(inputs[i]), i)
    for l in range(N_LAYERS):
        for i in range(N_QUBITS):
            s = _q_apply_1q(s, _q_ry(weights[l, i, 0]), i)
            s = _q_apply_1q(s, _q_rz(weights[l, i, 1]), i)
        for i in range(N_QUBITS):
            for j in range(i + 1, N_QUBITS):
                s = _q_cnot(s, i, j)
    probs = (s.real ** 2 + s.imag ** 2).reshape((2,) * N_QUBITS)
    evs = []
    for i in range(N_QUBITS):
        p = jnp.moveaxis(probs, i, 0)
        evs.append(jnp.sum(p[0]) - jnp.sum(p[1]))
    return jnp.stack(evs)


def kernel(x, edge_index, batch, W1, b1, W2, b2, Wp, bp, input_scale,
           quantum_scale, qweights, Wc1, bc1, Wc2, bc2):
    xp = jnp.zeros((NPAD, D_FEAT), jnp.float32).at[:N_NODES].set(x)
    pad_ids = (jnp.arange(EP - E, dtype=jnp.int32) % (NPAD - N_NODES)
               ) + N_NODES
    srcp = jnp.concatenate([edge_index[0], pad_ids]).reshape(NW, NCHUNK, CH)
    dstp = jnp.concatenate([edge_index[1], pad_ids]).reshape(NW, NCHUNK, CH)
    batchp = jnp.full((1, NPAD), N_GRAPHS, jnp.int32).at[0, :N_NODES].set(
        batch)

    degp = _deg_call(dstp)
    g1 = _tc1(xp, W1, degp)
    acc1 = _scat32(g1, srcp, dstp)
    g2 = _tc2(acc1, g1, degp, b1.reshape(1, 32), W2)
    acc2 = _scat16(g2, srcp, dstp)
    pooled = _tc3(acc2, g2, degp, b2.reshape(1, 16), batchp)

    q_in = jnp.tanh(pooled @ Wp + bp) * input_scale
    q_out = jax.vmap(_q_circuit, in_axes=(0, None))(q_in, qweights)
    q_out = q_out.astype(jnp.float32) * quantum_scale
    hid = jax.nn.relu(q_out @ Wc1 + bc1)
    return hid @ Wc2 + bc2


# speedup vs baseline: 45.8743x; 1.0669x over previous
import functools

import numpy as np
import jax
import jax.numpy as jnp
from jax import lax
from jax.experimental import pallas as pl
from jax.experimental.pallas import tpu as pltpu
from jax.experimental.pallas import tpu_sc as plsc

N_NODES = 10000
D_FEAT = 128
N_GRAPHS = 128
N_QUBITS = 4
N_LAYERS = 2

NC, NS, LANES = 2, 16, 16
NW = NC * NS
NPAD = 10240
RPT = NPAD // NS
E = 320000
CH = 128
EPW = 10240
NCHUNK = EPW // CH
EP = NW * EPW
NBUF = 8
DEG_NBUF = 4


def _zero_block(zb, d):
    @pl.loop(0, CH)
    def _(r):
        for j in range(d // LANES):
            zb[r, pl.ds(j * LANES, LANES)] = jnp.zeros((LANES,), jnp.float32)


def _fill_ones(zb, d):
    @pl.loop(0, CH)
    def _(r):
        for j in range(d // LANES):
            zb[r, pl.ds(j * LANES, LANES)] = jnp.ones((LANES,), jnp.float32)


_MESH = plsc.VectorSubcoreMesh(
    core_axis_name="c", subcore_axis_name="s", num_cores=NC, num_subcores=NS)
_SC_PARAMS = pltpu.CompilerParams(use_tc_tiling_on_sc=False)


def _make_deg_kernel():
    d = 16

    @functools.partial(
        pl.kernel,
        out_type=jax.ShapeDtypeStruct((NC, NPAD, d), jnp.float32),
        mesh=_MESH,
        compiler_params=_SC_PARAMS,
        scratch_types=[
            pltpu.VMEM((NCHUNK, CH), jnp.int32),
            pltpu.VMEM((CH, d), jnp.float32),
            pltpu.VMEM((CH, d), jnp.float32),
            pltpu.VMEM_SHARED((NPAD, d), jnp.float32),
        ] + [pltpu.SemaphoreType.DMA] * DEG_NBUF,
    )
    def deg_kernel(dst_hbm, out_hbm, dst_v, ones_v, zb_v, acc_sh, *sems):
        c = lax.axis_index("c")
        s = lax.axis_index("s")
        wid = c * NS + s
        _zero_block(zb_v, d)
        _fill_ones(ones_v, d)
        for k in range(RPT // CH):
            pltpu.sync_copy(zb_v, acc_sh.at[pl.ds(s * RPT + k * CH, CH)])
        pltpu.sync_copy(dst_hbm.at[wid], dst_v)
        plsc.subcore_barrier()

        @pl.loop(0, NCHUNK, step=DEG_NBUF)
        def _(t0):
            descs = []
            for b in range(DEG_NBUF):
                descs.append(pltpu.async_copy(
                    ones_v, acc_sh.at[dst_v.at[t0 + b]], sems[b], add=True))
            for b in range(DEG_NBUF):
                descs[b].wait()

        plsc.subcore_barrier()
        for k in range(RPT // CH):
            off = s * RPT + k * CH
            pltpu.sync_copy(acc_sh.at[pl.ds(off, CH)],
                            out_hbm.at[c, pl.ds(off, CH)])

    return deg_kernel


def _make_scatter_kernel(d):
    @functools.partial(
        pl.kernel,
        out_type=jax.ShapeDtypeStruct((NC, NPAD, d), jnp.float32),
        mesh=_MESH,
        compiler_params=_SC_PARAMS,
        scratch_types=[
            pltpu.VMEM((NCHUNK, CH), jnp.int32),
            pltpu.VMEM((NCHUNK, CH), jnp.int32),
            pltpu.VMEM((NBUF, CH, d), jnp.float32),
            pltpu.VMEM((CH, d), jnp.float32),
            pltpu.VMEM_SHARED((NPAD, d), jnp.float32),
        ] + [pltpu.SemaphoreType.DMA] * (2 * NBUF),
    )
    def scatter_kernel(g_hbm, src_hbm, dst_hbm, out_hbm,
                       src_v, dst_v, rows_v, zb_v, acc_sh, *sems):
        gsems, ssems = sems[:NBUF], sems[NBUF:]
        c = lax.axis_index("c")
        s = lax.axis_index("s")
        wid = c * NS + s
        _zero_block(zb_v, d)
        for k in range(RPT // CH):
            pltpu.sync_copy(zb_v, acc_sh.at[pl.ds(s * RPT + k * CH, CH)])
        pltpu.sync_copy(src_hbm.at[wid], src_v)
        pltpu.sync_copy(dst_hbm.at[wid], dst_v)
        plsc.subcore_barrier()

        @pl.loop(0, NCHUNK, step=NBUF)
        def _(t0):
            gds = []
            for b in range(NBUF):
                gds.append(pltpu.async_copy(
                    g_hbm.at[src_v.at[t0 + b]], rows_v.at[b], gsems[b]))
            sds = []
            for b in range(NBUF):
                gds[b].wait()
                sds.append(pltpu.async_copy(
                    rows_v.at[b], acc_sh.at[dst_v.at[t0 + b]], ssems[b],
                    add=True))
            for b in range(NBUF):
                sds[b].wait()

        plsc.subcore_barrier()
        for k in range(RPT // CH):
            off = s * RPT + k * CH
            pltpu.sync_copy(acc_sh.at[pl.ds(off, CH)],
                            out_hbm.at[c, pl.ds(off, CH)])

    return scatter_kernel


_deg_call = _make_deg_kernel()
_scat32 = _make_scatter_kernel(32)
_scat16 = _make_scatter_kernel(16)


def _dis_from(degp_ref):
    deg = degp_ref[0, :, 0:1] + degp_ref[1, :, 0:1] + 1.0
    return lax.rsqrt(deg)


def _tc1_body(x_ref, w1_ref, degp_ref, g1_ref):
    dis = _dis_from(degp_ref)
    h = jnp.dot(x_ref[...], w1_ref[...], preferred_element_type=jnp.float32)
    g1_ref[...] = h * dis


def _tc2_body(accp_ref, g1_ref, degp_ref, b1_ref, w2_ref, g2_ref):
    dis = _dis_from(degp_ref)
    z = (accp_ref[0] + accp_ref[1] + g1_ref[...]) * dis + b1_ref[...]
    h1 = jnp.maximum(z, 0.0)
    g2_ref[...] = jnp.dot(h1, w2_ref[...],
                          preferred_element_type=jnp.float32) * dis


def _tc3_body(accp_ref, g2_ref, degp_ref, b2_ref, batch_ref, pooled_ref):
    dis = _dis_from(degp_ref)
    z = (accp_ref[0] + accp_ref[1] + g2_ref[...]) * dis + b2_ref[...]
    h2 = jnp.maximum(z, 0.0)
    gi = lax.broadcasted_iota(jnp.int32, (N_GRAPHS, NPAD), 0)
    onehot = (batch_ref[...] == gi).astype(jnp.float32)
    sums = jnp.dot(onehot, h2, preferred_element_type=jnp.float32,
                   precision=lax.Precision.HIGHEST)
    counts = jnp.sum(onehot, axis=1, keepdims=True)
    pooled_ref[...] = sums / jnp.maximum(counts, 1.0)


_tc1 = pl.pallas_call(
    _tc1_body, out_shape=jax.ShapeDtypeStruct((NPAD, 32), jnp.float32))
_tc2 = pl.pallas_call(
    _tc2_body, out_shape=jax.ShapeDtypeStruct((NPAD, 16), jnp.float32))
_tc3 = pl.pallas_call(
    _tc3_body, out_shape=jax.ShapeDtypeStruct((N_GRAPHS, 16), jnp.float32))


def _q_apply_1q(state, u, wire):
    s = state.reshape((2,) * N_QUBITS)
    s = jnp.moveaxis(s, wire, 0)
    s = jnp.tensordot(u, s, axes=([1], [0]))
    s = jnp.moveaxis(s, 0, wire)
    return s.reshape(-1)


def _q_cnot(state, c, t):
    s = state.reshape((2,) * N_QUBITS)
    s = jnp.moveaxis(s, (c, t), (0, 1))
    s = jnp.stack([s[0], s[1][::-1]], axis=0)
    s = jnp.moveaxis(s, (0, 1), (c, t))
    return s.reshape(-1)


def _q_ry(t):
    c = jnp.cos(t / 2).astype(jnp.complex64)
    sn = jnp.sin(t / 2).astype(jnp.complex64)
    return jnp.stack([jnp.stack([c, -sn]), jnp.stack([sn, c])])


def _q_rz(t):
    p = jnp.exp(-0.5j * t).astype(jnp.complex64)
    m = jnp.exp(0.5j * t).astype(jnp.complex64)
    z = jnp.zeros((), jnp.complex64)
    return jnp.stack([jnp.stack([p, z]), jnp.stack([z, m])])


def _q_circuit(inputs, weights):
    s = jnp.zeros((2 ** N_QUBITS,), jnp.complex64).at[0].set(1.0)
    for i in range(N_QUBITS):
        s = _q_apply_1q(s, _q_ry(inputs[i]), i)
    for l in range(N_LAYERS):
        for i in range(N_QUBITS):
            s = _q_apply_1q(s, _q_ry(weights[l, i, 0]), i)
            s = _q_apply_1q(s, _q_rz(weights[l, i, 1]), i)
        for i in range(N_QUBITS):
            for j in range(i + 1, N_QUBITS):
                s = _q_cnot(s, i, j)
    probs = (s.real ** 2 + s.imag ** 2).reshape((2,) * N_QUBITS)
    evs = []
    for i in range(N_QUBITS):
        p = jnp.moveaxis(probs, i, 0)
        evs.append(jnp.sum(p[0]) - jnp.sum(p[1]))
    return jnp.stack(evs)


def kernel(x, edge_index, batch, W1, b1, W2, b2, Wp, bp, input_scale,
           quantum_scale, qweights, Wc1, bc1, Wc2, bc2):
    xp = jnp.zeros((NPAD, D_FEAT), jnp.float32).at[:N_NODES].set(x)
    pad_ids = (jnp.arange(EP - E, dtype=jnp.int32) % (NPAD - N_NODES)
               ) + N_NODES
    srcp = jnp.concatenate([edge_index[0], pad_ids]).reshape(NW, NCHUNK, CH)
    dstp = jnp.concatenate([edge_index[1], pad_ids]).reshape(NW, NCHUNK, CH)
    batchp = jnp.full((1, NPAD), N_GRAPHS, jnp.int32).at[0, :N_NODES].set(
        batch)

    degp = _deg_call(dstp)
    g1 = _tc1(xp, W1, degp)
    acc1 = _scat32(g1, srcp, dstp)
    g2 = _tc2(acc1, g1, degp, b1.reshape(1, 32), W2)
    acc2 = _scat16(g2, srcp, dstp)
    pooled = _tc3(acc2, g2, degp, b2.reshape(1, 16), batchp)

    q_in = jnp.tanh(pooled @ Wp + bp) * input_scale
    q_out = jax.vmap(_q_circuit, in_axes=(0, None))(q_in, qweights)
    q_out = q_out.astype(jnp.float32) * quantum_scale
    hid = jax.nn.relu(q_out @ Wc1 + bc1)
    return hid @ Wc2 + bc2
